# baseline (speedup 1.0000x reference)
import jax
import jax.numpy as jnp
from jax.experimental import pallas as pl

N = 4096
BM = 512


def _leaky(x):
    return jnp.where(x >= 0, x, 0.25 * x)


def _proj_kernel(h_ref, w_ref, o_ref):
    o_ref[...] = jnp.dot(h_ref[...], w_ref[...],
                         preferred_element_type=jnp.float32)


def _layer_fused_kernel(g_ref, s_ref, b_ref, w_ref, o_ref):
    t = jnp.dot(g_ref[...], s_ref[...], preferred_element_type=jnp.float32)
    t = _leaky(t + b_ref[...])
    o_ref[...] = jnp.dot(t, w_ref[...], preferred_element_type=jnp.float32)


def _layer_last_kernel(g_ref, s_ref, b_ref, o_ref):
    t = jnp.dot(g_ref[...], s_ref[...], preferred_element_type=jnp.float32)
    o_ref[...] = _leaky(t + b_ref[...])


def _decoder_kernel(hr_ref, tw_ref, hd_ref, o_ref):
    a = jnp.dot(hr_ref[...], tw_ref[...], preferred_element_type=jnp.float32)
    o_ref[...] = jax.lax.dot_general(
        a, hd_ref[...], (((1,), (1,)), ((), ())),
        preferred_element_type=jnp.float32)


def _layer(G, S, b, W):
    hid = S.shape[1]
    b2d = b.reshape(1, hid)
    if W is None:
        return pl.pallas_call(
            _layer_last_kernel,
            grid=(N // BM,),
            in_specs=[
                pl.BlockSpec((BM, N), lambda i: (i, 0)),
                pl.BlockSpec((N, hid), lambda i: (0, 0)),
                pl.BlockSpec((1, hid), lambda i: (0, 0)),
            ],
            out_specs=pl.BlockSpec((BM, hid), lambda i: (i, 0)),
            out_shape=jax.ShapeDtypeStruct((N, hid), jnp.float32),
        )(G, S, b2d)
    return pl.pallas_call(
        _layer_fused_kernel,
        grid=(N // BM,),
        in_specs=[
            pl.BlockSpec((BM, N), lambda i: (i, 0)),
            pl.BlockSpec((N, hid), lambda i: (0, 0)),
            pl.BlockSpec((1, hid), lambda i: (0, 0)),
            pl.BlockSpec((hid, hid), lambda i: (0, 0)),
        ],
        out_specs=pl.BlockSpec((BM, hid), lambda i: (i, 0)),
        out_shape=jax.ShapeDtypeStruct((N, hid), jnp.float32),
    )(G, S, b2d, W)


def kernel(H, G, W1, b1, W2, b2, W3, b3, train_W, drug_num, target_num):
    n, in_dim = H.shape
    hid = W1.shape[1]

    S1 = pl.pallas_call(
        _proj_kernel,
        grid=(n // BM,),
        in_specs=[
            pl.BlockSpec((BM, in_dim), lambda i: (i, 0)),
            pl.BlockSpec((in_dim, hid), lambda i: (0, 0)),
        ],
        out_specs=pl.BlockSpec((BM, hid), lambda i: (i, 0)),
        out_shape=jax.ShapeDtypeStruct((n, hid), jnp.float32),
    )(H, W1)

    S2 = _layer(G, S1, b1, W2)
    S3 = _layer(G, S2, b2, W3)
    h3 = _layer(G, S3, b3, None)

    d = n // 2
    t = n - d
    HR = jax.lax.dynamic_slice_in_dim(h3, drug_num - d, d)
    HD = jax.lax.dynamic_slice_in_dim(h3, drug_num + target_num - t, t)

    out = pl.pallas_call(
        _decoder_kernel,
        grid=(d // BM,),
        in_specs=[
            pl.BlockSpec((BM, hid), lambda i: (i, 0)),
            pl.BlockSpec((hid, hid), lambda i: (0, 0)),
            pl.BlockSpec((t, hid), lambda i: (0, 0)),
        ],
        out_specs=pl.BlockSpec((BM, t), lambda i: (i, 0)),
        out_shape=jax.ShapeDtypeStruct((d, t), jnp.float32),
    )(HR, train_W, HD)
    return out
